# baseline (speedup 1.0000x reference)
import functools

import jax
import jax.numpy as jnp
from jax import lax
from jax.experimental import pallas as pl
from jax.experimental.pallas import tpu as pltpu
from jax.experimental.pallas import tpu_sc as plsc

NC, NS = 2, 16
NW = NC * NS
CSZ = 50
NBUF = 12
GLEAD = 6
RU = 5


def _make_embed(BL, D, L, n_chunks):
    mesh = plsc.VectorSubcoreMesh(
        core_axis_name="c", subcore_axis_name="s", num_cores=NC, num_subcores=NS
    )
    rows_per_w = BL // NW
    nvec = D // 16

    @functools.partial(
        pl.kernel,
        out_type=jax.ShapeDtypeStruct((BL, D), jnp.float32),
        mesh=mesh,
        scratch_types=[
            pltpu.VMEM((n_chunks, CSZ), jnp.int32),
            pltpu.VMEM((L, D), jnp.float32),
            pltpu.VMEM((NBUF, CSZ, D), jnp.float32),
            pltpu.SemaphoreType.DMA((NBUF,)),
            pltpu.SemaphoreType.DMA((NBUF,)),
        ],
        compiler_params=pltpu.CompilerParams(use_tc_tiling_on_sc=False),
    )
    def k(idx_hbm, word_hbm, pos_hbm, out_hbm, idx_v, pos_v, bufs, gsem, osem):
        wid = lax.axis_index("s") * NC + lax.axis_index("c")
        base = wid * rows_per_w
        pltpu.sync_copy(idx_hbm.at[wid], idx_v)
        pltpu.sync_copy(pos_hbm, pos_v)

        def issue_g(c, b):
            pltpu.async_copy(word_hbm.at[idx_v.at[c]], bufs.at[b], gsem.at[b])

        def wait_g(b):
            pltpu.make_async_copy(
                word_hbm.at[idx_v.at[0]], bufs.at[b], gsem.at[b]
            ).wait()

        def issue_o(c, b):
            pltpu.async_copy(
                bufs.at[b], out_hbm.at[pl.ds(base + c * CSZ, CSZ)], osem.at[b]
            )

        def wait_o(b):
            pltpu.make_async_copy(
                bufs.at[b], out_hbm.at[pl.ds(base, CSZ)], osem.at[b]
            ).wait()

        def add_pos(c, b):
            off = (c * CSZ) % L

            def row(r0, _):
                for u in range(RU):
                    r = r0 * RU + u
                    for j in range(nvec):
                        x = pos_v[off + r, pl.ds(j * 16, 16)]
                        plsc.addupdate(bufs.at[b, r, pl.ds(j * 16, 16)], x)
                return 0

            lax.fori_loop(0, CSZ // RU, row, 0)

        for b in range(GLEAD):
            issue_g(b, b)

        def chunk(c, b):
            wait_g(b)
            add_pos(c, b)
            issue_o(c, b)
            b2 = (b + GLEAD) % NBUF

            @pl.when(jnp.logical_and(c >= NBUF - GLEAD, c + GLEAD < n_chunks))
            def _():
                wait_o(b2)

            @pl.when(c + GLEAD < n_chunks)
            def _():
                issue_g(c + GLEAD, b2)

        n_iters = -(-n_chunks // NBUF)

        def body(h, _):
            for r in range(NBUF):
                c = h * NBUF + r

                @pl.when(c < n_chunks)
                def _():
                    chunk(c, r)

            return 0

        lax.fori_loop(0, n_iters, body, 0)
        for b in range(NBUF):
            wait_o(b)

    return k


def kernel(sequence, sequence_char, word_table, pos_table):
    del sequence_char
    B, L = sequence.shape
    D = word_table.shape[1]
    BL = B * L
    rows_per_w = BL // NW
    n_chunks = rows_per_w // CSZ
    idx = sequence.astype(jnp.int32).reshape(NW, n_chunks, CSZ)
    out = _make_embed(BL, D, L, n_chunks)(
        idx, word_table, pos_table[:L].astype(jnp.float32)
    )
    return out.reshape(B, L, D)


# speedup vs baseline: 1.0329x; 1.0329x over previous
import functools

import jax
import jax.numpy as jnp
from jax import lax
from jax.experimental import pallas as pl
from jax.experimental.pallas import tpu as pltpu
from jax.experimental.pallas import tpu_sc as plsc

NC, NS = 2, 16
NW = NC * NS
CSZ = 100
NBUF = 6
GLEAD = 3
RU = 5


def _make_embed(BL, D, L, n_chunks):
    mesh = plsc.VectorSubcoreMesh(
        core_axis_name="c", subcore_axis_name="s", num_cores=NC, num_subcores=NS
    )
    rows_per_w = BL // NW
    nvec = D // 16

    @functools.partial(
        pl.kernel,
        out_type=jax.ShapeDtypeStruct((BL, D), jnp.float32),
        mesh=mesh,
        scratch_types=[
            pltpu.VMEM((n_chunks, CSZ), jnp.int32),
            pltpu.VMEM((L, D), jnp.float32),
            pltpu.VMEM((NBUF, CSZ, D), jnp.float32),
            pltpu.SemaphoreType.DMA((NBUF,)),
            pltpu.SemaphoreType.DMA((NBUF,)),
        ],
        compiler_params=pltpu.CompilerParams(use_tc_tiling_on_sc=False),
    )
    def k(idx_hbm, word_hbm, pos_hbm, out_hbm, idx_v, pos_v, bufs, gsem, osem):
        wid = lax.axis_index("s") * NC + lax.axis_index("c")
        base = wid * rows_per_w
        pltpu.sync_copy(idx_hbm.at[wid], idx_v)

        def issue_g(c, b):
            pltpu.async_copy(word_hbm.at[idx_v.at[c]], bufs.at[b], gsem.at[b])

        def wait_g(b):
            pltpu.make_async_copy(
                word_hbm.at[idx_v.at[0]], bufs.at[b], gsem.at[b]
            ).wait()

        def issue_o(c, b):
            pltpu.async_copy(
                bufs.at[b], out_hbm.at[pl.ds(base + c * CSZ, CSZ)], osem.at[b]
            )

        def wait_o(b):
            pltpu.make_async_copy(
                bufs.at[b], out_hbm.at[pl.ds(base, CSZ)], osem.at[b]
            ).wait()

        def add_pos(c, b):
            off = (c * CSZ) % L

            def row(r0, _):
                for u in range(RU):
                    r = r0 * RU + u
                    for j in range(nvec):
                        x = pos_v[off + r, pl.ds(j * 16, 16)]
                        plsc.addupdate(bufs.at[b, r, pl.ds(j * 16, 16)], x)
                return 0

            lax.fori_loop(0, CSZ // RU, row, 0)

        for b in range(GLEAD):
            issue_g(b, b)
        pltpu.sync_copy(pos_hbm, pos_v)

        def chunk(c, b):
            wait_g(b)
            add_pos(c, b)
            issue_o(c, b)
            b2 = (b + GLEAD) % NBUF

            @pl.when(jnp.logical_and(c >= NBUF - GLEAD, c + GLEAD < n_chunks))
            def _():
                wait_o(b2)

            @pl.when(c + GLEAD < n_chunks)
            def _():
                issue_g(c + GLEAD, b2)

        n_iters = -(-n_chunks // NBUF)

        def body(h, _):
            for r in range(NBUF):
                c = h * NBUF + r

                @pl.when(c < n_chunks)
                def _():
                    chunk(c, r)

            return 0

        lax.fori_loop(0, n_iters, body, 0)
        for b in range(NBUF):
            wait_o(b)

    return k


def kernel(sequence, sequence_char, word_table, pos_table):
    del sequence_char
    B, L = sequence.shape
    D = word_table.shape[1]
    BL = B * L
    rows_per_w = BL // NW
    n_chunks = rows_per_w // CSZ
    idx = sequence.astype(jnp.int32).reshape(NW, n_chunks, CSZ)
    out = _make_embed(BL, D, L, n_chunks)(
        idx, word_table, pos_table[:L].astype(jnp.float32)
    )
    return out.reshape(B, L, D)
